# baseline (speedup 1.0000x reference)
import functools

import jax
import jax.numpy as jnp
from jax import lax
from jax.experimental import pallas as pl
from jax.experimental.pallas import tpu as pltpu
from jax.experimental.pallas import tpu_sc as plsc

_B, _S, _D = 4, 2048, 1024
_NC, _NS = 2, 16
_NW = _NC * _NS
_ROWS_W = _S // _NW
_CR = 8
_NCH = _ROWS_W // _CR
_NX = 6
_NIT = _NCH * _B

_scratch = (
    [pltpu.VMEM((_CR, _D), jnp.float32) for _ in range(2)]
    + [pltpu.VMEM((_CR, _D), jnp.float32) for _ in range(_NX)]
    + [pltpu.SemaphoreType.DMA for _ in range(2 + 2 * _NX)]
)


@functools.partial(
    pl.kernel,
    out_type=jax.ShapeDtypeStruct((_B, _S, _D), jnp.float32),
    mesh=plsc.VectorSubcoreMesh(core_axis_name="c", subcore_axis_name="s"),
    scratch_types=_scratch,
    compiler_params=pltpu.CompilerParams(
        use_tc_tiling_on_sc=True, skip_device_barrier=True
    ),
)
def _pos_enc(x_hbm, t_hbm, out_hbm, *sc):
    tb = list(sc[0:2])
    xb = list(sc[2 : 2 + _NX])
    tsem = list(sc[2 + _NX : 4 + _NX])
    xsem = list(sc[4 + _NX : 4 + _NX + _NX])
    osem = list(sc[4 + 2 * _NX : 4 + 3 * _NX])

    wid = lax.axis_index("s") * _NC + lax.axis_index("c")
    base = wid * _ROWS_W

    def s0_of(ci):
        return base + ci * _CR

    def start_tab(ci):
        return pltpu.async_copy(
            t_hbm.at[pl.ds(s0_of(ci), _CR), :], tb[ci % 2], tsem[ci % 2]
        )

    def start_in(i):
        ci, b, sl = i // _B, i % _B, i % _NX
        return pltpu.async_copy(
            x_hbm.at[b, pl.ds(s0_of(ci), _CR), :], xb[sl], xsem[sl]
        )

    def start_out(i):
        ci, b, sl = i // _B, i % _B, i % _NX
        return pltpu.async_copy(
            xb[sl], out_hbm.at[b, pl.ds(s0_of(ci), _CR), :], osem[sl]
        )

    tab_d = {0: start_tab(0), 1: start_tab(1)}
    in_d = {i: start_in(i) for i in range(_NX - 2)}
    out_d = {}
    for i in range(_NIT):
        ci = i // _B
        if i - 2 >= 0:
            out_d.pop(i - 2).wait()
        if i + _NX - 2 < _NIT:
            in_d[i + _NX - 2] = start_in(i + _NX - 2)
        in_d.pop(i).wait()
        if i % _B == 0:
            tab_d.pop(ci).wait()

        sl = i % _NX
        tref = tb[ci % 2]

        @pl.loop(0, _CR)
        def _row(r):
            @plsc.parallel_loop(0, _D, step=16, unroll=8)
            def _vec(c):
                csl = pl.ds(c, 16)
                plsc.addupdate(xb[sl].at[r, csl], tref[r, csl])

        out_d[i] = start_out(i)
        if i % _B == _B - 1 and ci + 2 < _NCH:
            tab_d[ci + 2] = start_tab(ci + 2)

    for i in range(max(0, _NIT - 2), _NIT):
        out_d.pop(i).wait()


def kernel(x, table):
    return _pos_enc(x, table)


# speedup vs baseline: 1.0066x; 1.0066x over previous
import functools

import jax
import jax.numpy as jnp
from jax import lax
from jax.experimental import pallas as pl
from jax.experimental.pallas import tpu as pltpu
from jax.experimental.pallas import tpu_sc as plsc

_B, _S, _D = 4, 2048, 1024
_NC, _NS = 2, 16
_NW = _NC * _NS
_ROWS_W = _S // _NW
_CR = 16
_NCH = _ROWS_W // _CR
_NX = 4
_NIT = _NCH * _B

_scratch = (
    [pltpu.VMEM((_CR, _D), jnp.float32) for _ in range(2)]
    + [pltpu.VMEM((_CR, _D), jnp.float32) for _ in range(_NX)]
    + [pltpu.SemaphoreType.DMA for _ in range(2 + 2 * _NX)]
)


@functools.partial(
    pl.kernel,
    out_type=jax.ShapeDtypeStruct((_B, _S, _D), jnp.float32),
    mesh=plsc.VectorSubcoreMesh(core_axis_name="c", subcore_axis_name="s"),
    scratch_types=_scratch,
    compiler_params=pltpu.CompilerParams(use_tc_tiling_on_sc=True),
)
def _pos_enc(x_hbm, t_hbm, out_hbm, *sc):
    tb = list(sc[0:2])
    xb = list(sc[2 : 2 + _NX])
    tsem = list(sc[2 + _NX : 4 + _NX])
    xsem = list(sc[4 + _NX : 4 + _NX + _NX])
    osem = list(sc[4 + 2 * _NX : 4 + 3 * _NX])

    wid = lax.axis_index("s") * _NC + lax.axis_index("c")
    base = wid * _ROWS_W

    def s0_of(ci):
        return base + ci * _CR

    def start_tab(ci):
        return pltpu.async_copy(
            t_hbm.at[pl.ds(s0_of(ci), _CR), :], tb[ci % 2], tsem[ci % 2]
        )

    def start_in(i):
        ci, b, sl = i // _B, i % _B, i % _NX
        return pltpu.async_copy(
            x_hbm.at[b, pl.ds(s0_of(ci), _CR), :], xb[sl], xsem[sl]
        )

    def start_out(i):
        ci, b, sl = i // _B, i % _B, i % _NX
        return pltpu.async_copy(
            xb[sl], out_hbm.at[b, pl.ds(s0_of(ci), _CR), :], osem[sl]
        )

    tab_d = {0: start_tab(0), 1: start_tab(1)}
    in_d = {i: start_in(i) for i in range(_NX - 2)}
    out_d = {}
    for i in range(_NIT):
        ci = i // _B
        if i - 2 >= 0:
            out_d.pop(i - 2).wait()
        if i + _NX - 2 < _NIT:
            in_d[i + _NX - 2] = start_in(i + _NX - 2)
        in_d.pop(i).wait()
        if i % _B == 0:
            tab_d.pop(ci).wait()

        sl = i % _NX
        tref = tb[ci % 2]

        @pl.loop(0, _CR)
        def _row(r):
            @plsc.parallel_loop(0, _D, step=16, unroll=8)
            def _vec(c):
                csl = pl.ds(c, 16)
                plsc.addupdate(xb[sl].at[r, csl], tref[r, csl])

        out_d[i] = start_out(i)
        if i % _B == _B - 1 and ci + 2 < _NCH:
            tab_d[ci + 2] = start_tab(ci + 2)

    for i in range(max(0, _NIT - 2), _NIT):
        out_d.pop(i).wait()


def kernel(x, table):
    return _pos_enc(x, table)
